# baseline (speedup 1.0000x reference)
import functools

import jax
import jax.numpy as jnp
from jax import lax
from jax.experimental import pallas as pl
from jax.experimental.pallas import tpu as pltpu
from jax.experimental.pallas import tpu_sc as plsc

N = 10000
E = 320000
F = 128
H = 64
G = 16
EPS = 1e-5

NC, NS = 2, 16
NW = NC * NS
NP = 10240
EP = 327680
BLK = 128
NBLK = EP // (NW * BLK)
DPT = E // NW
RB = 2048
NRB = NP // RB

@functools.cache
def _sc_params():
    import dataclasses
    cp = pltpu.CompilerParams()
    if "needs_layout_passes" in pltpu.CompilerParams.__dataclass_fields__:
        cp = dataclasses.replace(cp, needs_layout_passes=False)
    if "use_tc_tiling_on_sc" in pltpu.CompilerParams.__dataclass_fields__:
        cp = dataclasses.replace(cp, use_tc_tiling_on_sc=False)
    return cp


@functools.cache
def _mesh():
    return plsc.VectorSubcoreMesh(
        core_axis_name="c", subcore_axis_name="s",
        num_cores=NC, num_subcores=NS)


def _sc_deg_body(dst_hbm, idr_hbm, out_hbm, idx_v, hist_v, idr_v, zv, acc_sh,
                 sem):
    c = lax.axis_index("c")
    s = lax.axis_index("s")
    wid = s * NC + c
    zero16 = jnp.zeros((16,), jnp.float32)

    @pl.loop(0, 640)
    def _zero_hist(i):
        hist_v[i, :] = zero16

    @pl.loop(0, 40)
    def _zero_zv(i):
        zv[i, :] = zero16

    pltpu.sync_copy(zv, acc_sh.at[pl.ds(s * 40, 40)])
    pltpu.sync_copy(dst_hbm.at[pl.ds(wid * DPT, DPT)], idx_v)
    pltpu.sync_copy(idr_hbm, idr_v)
    plsc.subcore_barrier()

    ones16 = jnp.ones((16,), jnp.float32)

    @pl.loop(0, DPT // 16)
    def _hist(i):
        nid = idx_v[pl.ds(i * 16, 16)]
        row = lax.shift_right_logical(nid, 4)
        col = lax.bitwise_and(nid, 15)
        plsc.addupdate_scatter(hist_v, [row, col], ones16)

    @pl.loop(0, 5)
    def _merge(j):
        pltpu.sync_copy(hist_v.at[pl.ds(j * 128, 128)],
                        acc_sh.at[idr_v.at[j]], add=True)

    plsc.subcore_barrier()
    pltpu.sync_copy(acc_sh.at[pl.ds(s * 40, 40)],
                    out_hbm.at[pl.ds(c * 640 + s * 40, 40)])


@jax.jit
def _sc_deg(dst, idr):
    return pl.kernel(
        _sc_deg_body,
        out_type=jax.ShapeDtypeStruct((2 * 640, 16), jnp.float32),
        mesh=_mesh(),
        compiler_params=_sc_params(),
        scratch_types=[
            pltpu.VMEM((DPT,), jnp.int32),
            pltpu.VMEM((640, 16), jnp.float32),
            pltpu.VMEM((5, 128), jnp.int32),
            pltpu.VMEM((40, 16), jnp.float32),
            pltpu.VMEM_SHARED((640, 16), jnp.float32),
            pltpu.SemaphoreType.DMA,
        ],
    )(dst, idr)


HH = H // 2


def _sc_agg_body(tabA, tabB, srcp_hbm, dstp_hbm, outA, outB,
                 sidx_v, didx_v, rows, semg, sems, acc_sh, tab_sh):
    c = lax.axis_index("c")
    s = lax.axis_index("s")
    wid = s * NC + c
    rpt = NP // NS
    NB = 4

    pltpu.sync_copy(srcp_hbm.at[wid], sidx_v)
    pltpu.sync_copy(dstp_hbm.at[wid], didx_v)

    for tab_hbm, out_hbm in ((tabA, outA), (tabB, outB)):
        pltpu.sync_copy(tab_hbm.at[pl.ds(s * rpt, rpt)],
                        tab_sh.at[pl.ds(s * rpt, rpt)])

        @pl.loop(0, BLK)
        def _zrow(i):
            @pl.loop(0, HH, step=16)
            def _zcol(k):
                rows[0, i, pl.ds(k, 16)] = jnp.zeros((16,), jnp.float32)

        @pl.loop(0, rpt, step=BLK)
        def _zacc(r):
            pltpu.sync_copy(rows.at[0], acc_sh.at[pl.ds(s * rpt + r, BLK)])

        plsc.subcore_barrier()

        def gather(k, b):
            pltpu.async_copy(tab_sh.at[sidx_v.at[k]], rows.at[b], semg[b])

        def wait_gather(k, b):
            pltpu.make_async_copy(tab_sh.at[sidx_v.at[k]], rows.at[b],
                                  semg[b]).wait()

        def scat(k, b):
            pltpu.async_copy(rows.at[b], acc_sh.at[didx_v.at[k]], sems[b],
                             add=True)

        def wait_scat(k, b):
            pltpu.make_async_copy(rows.at[b], acc_sh.at[didx_v.at[k]],
                                  sems[b]).wait()

        gather(0, 0)
        gather(1, 1)

        @pl.loop(0, NBLK, step=NB)
        def _edges(j):
            for b in range(NB):
                k = j + b
                gb = (b + 2) % NB

                @pl.when(k - 2 >= 0)
                def _():
                    wait_scat(k - 2, gb)

                @pl.when(k + 2 < NBLK)
                def _():
                    gather(k + 2, gb)

                wait_gather(k, b)
                scat(k, b)

        for k in (NBLK - 2, NBLK - 1):
            wait_scat(k, k % NB)

        plsc.subcore_barrier()
        pltpu.sync_copy(acc_sh.at[pl.ds(s * rpt, rpt)],
                        out_hbm.at[pl.ds(c * NP + s * rpt, rpt)])
        plsc.subcore_barrier()


@jax.jit
def _sc_agg(tabA, tabB, srcp, dstp):
    return pl.kernel(
        _sc_agg_body,
        out_type=(jax.ShapeDtypeStruct((2 * NP, HH), jnp.float32),
                  jax.ShapeDtypeStruct((2 * NP, HH), jnp.float32)),
        mesh=_mesh(),
        compiler_params=_sc_params(),
        scratch_types=[
            pltpu.VMEM((NBLK, BLK), jnp.int32),
            pltpu.VMEM((NBLK, BLK), jnp.int32),
            pltpu.VMEM((4, BLK, HH), jnp.float32),
            [pltpu.SemaphoreType.DMA] * 4,
            [pltpu.SemaphoreType.DMA] * 4,
            pltpu.VMEM_SHARED((NP, HH), jnp.float32),
            pltpu.VMEM_SHARED((NP, HH), jnp.float32),
        ],
    )(tabA, tabB, srcp, dstp)


def _mm_body(x_ref, w_ref, o_ref):
    o_ref[...] = jnp.dot(x_ref[...], w_ref[...],
                         preferred_element_type=jnp.float32)


def _tc_mm(x, w):
    m, k = x.shape
    _, n = w.shape
    return pl.pallas_call(
        _mm_body,
        grid=(m // RB,),
        in_specs=[pl.BlockSpec((RB, k), lambda i: (i, 0)),
                  pl.BlockSpec((k, n), lambda i: (0, 0))],
        out_specs=pl.BlockSpec((RB, n), lambda i: (i, 0)),
        out_shape=jax.ShapeDtypeStruct((m, n), jnp.float32),
    )(x, w)


def _dinv_body(dp_ref, o_ref):
    deg = dp_ref[0] + dp_ref[1] + 1.0
    r = lax.broadcasted_iota(jnp.int32, (80, 128), 0)
    cidx = lax.broadcasted_iota(jnp.int32, (80, 128), 1)
    nid = r * 128 + cidx
    o_ref[...] = jnp.where(nid < N, lax.rsqrt(deg), 0.0)


def _tc_dinv(degp):
    return pl.pallas_call(
        _dinv_body,
        out_shape=jax.ShapeDtypeStruct((80, 128), jnp.float32),
    )(degp)


def _scale_body(m_ref, d_ref, oa_ref, ob_ref):
    v = m_ref[...] * d_ref[...]
    oa_ref[...] = v[:, :HH]
    ob_ref[...] = v[:, HH:]


def _tc_scale(m, dcol):
    return pl.pallas_call(
        _scale_body,
        grid=(NRB,),
        in_specs=[pl.BlockSpec((RB, H), lambda i: (i, 0)),
                  pl.BlockSpec((RB, 1), lambda i: (i, 0))],
        out_specs=[pl.BlockSpec((RB, HH), lambda i: (i, 0)),
                   pl.BlockSpec((RB, HH), lambda i: (i, 0))],
        out_shape=[jax.ShapeDtypeStruct((NP, HH), jnp.float32),
                   jax.ShapeDtypeStruct((NP, HH), jnp.float32)],
    )(m, dcol)


def _accum_out(aa0, aa1, ab0, ab1, hpa, hpb, d, b, i):
    agg = jnp.concatenate([aa0 + aa1 + hpa, ab0 + ab1 + hpb], axis=1)
    v = agg * d + b
    rid = lax.broadcasted_iota(jnp.int32, (RB, 1), 0) + i * RB
    return jnp.where(rid < N, v, 0.0)


def _bn_coeffs(st_ref, prm_ref, grow, berow):
    mu = st_ref[0:1, :] * (1.0 / N)
    var = st_ref[1:2, :] * (1.0 / N) - mu * mu
    istd = lax.rsqrt(var + EPS)
    g = prm_ref[grow:grow + 1, :H]
    be = prm_ref[berow:berow + 1, :H]
    return mu, istd * g, be


def _mid_body(aa0_ref, aa1_ref, ab0_ref, ab1_ref, hpa_ref, hpb_ref, d_ref,
              prm_ref, w_ref, oa_ref, ob_ref, o1_ref, st_ref, *, brow):
    i = pl.program_id(0)

    @pl.when(i < NRB)
    def _():
        b = prm_ref[brow:brow + 1, :H]
        v = _accum_out(aa0_ref[...], aa1_ref[...], ab0_ref[...], ab1_ref[...],
                       hpa_ref[...], hpb_ref[...], d_ref[...], b, i)
        o1_ref[pl.ds(i * RB, RB), :] = v
        srow = jnp.sum(v, axis=0, keepdims=True)
        qrow = jnp.sum(v * v, axis=0, keepdims=True)
        st = jnp.concatenate(
            [srow, qrow, jnp.zeros((6, H), jnp.float32)], axis=0)

        @pl.when(i == 0)
        def _():
            st_ref[...] = st

        @pl.when(i > 0)
        def _():
            st_ref[...] = st_ref[...] + st

    @pl.when(i >= NRB)
    def _():
        mu, a, be = _bn_coeffs(st_ref, prm_ref, brow + 1, brow + 2)
        h = jnp.maximum((o1_ref[pl.ds((i - NRB) * RB, RB), :] - mu) * a + be,
                        0.0)
        v = jnp.dot(h, w_ref[...],
                    preferred_element_type=jnp.float32) * d_ref[...]
        oa_ref[...] = v[:, :HH]
        ob_ref[...] = v[:, HH:]


def _tc_mid(accpa, accpb, hpa, hpb, dcol, prm, w, brow):
    j = lambda i: jnp.where(i < NRB, i, i - NRB)
    jo = lambda i: jnp.where(i < NRB, 0, i - NRB)
    return pl.pallas_call(
        functools.partial(_mid_body, brow=brow),
        grid=(2 * NRB,),
        in_specs=[pl.BlockSpec((RB, HH), lambda i: (j(i), 0)),
                  pl.BlockSpec((RB, HH), lambda i: (NRB + j(i), 0)),
                  pl.BlockSpec((RB, HH), lambda i: (j(i), 0)),
                  pl.BlockSpec((RB, HH), lambda i: (NRB + j(i), 0)),
                  pl.BlockSpec((RB, HH), lambda i: (j(i), 0)),
                  pl.BlockSpec((RB, HH), lambda i: (j(i), 0)),
                  pl.BlockSpec((RB, 1), lambda i: (j(i), 0)),
                  pl.BlockSpec((8, 128), lambda i: (0, 0)),
                  pl.BlockSpec((H, H), lambda i: (0, 0))],
        out_specs=[pl.BlockSpec((RB, HH), lambda i: (jo(i), 0)),
                   pl.BlockSpec((RB, HH), lambda i: (jo(i), 0))],
        out_shape=[jax.ShapeDtypeStruct((NP, HH), jnp.float32),
                   jax.ShapeDtypeStruct((NP, HH), jnp.float32)],
        scratch_shapes=[pltpu.VMEM((NP, H), jnp.float32),
                        pltpu.VMEM((8, H), jnp.float32)],
    )(accpa, accpa, accpb, accpb, hpa, hpb, dcol, prm, w)


def _tail_body(aa0_ref, aa1_ref, ab0_ref, ab1_ref, hpa_ref, hpb_ref, d_ref,
               prm_ref, bt_ref, o_ref, o2_ref, st_ref, cnt_ref):
    i = pl.program_id(0)

    @pl.when(i < NRB)
    def _():
        b = prm_ref[3:4, :H]
        v = _accum_out(aa0_ref[...], aa1_ref[...], ab0_ref[...], ab1_ref[...],
                       hpa_ref[...], hpb_ref[...], d_ref[...], b, i)
        o2_ref[pl.ds(i * RB, RB), :] = v
        srow = jnp.sum(v, axis=0, keepdims=True)
        qrow = jnp.sum(v * v, axis=0, keepdims=True)
        st = jnp.concatenate(
            [srow, qrow, jnp.zeros((6, H), jnp.float32)], axis=0)

        @pl.when(i == 0)
        def _():
            st_ref[...] = st

        @pl.when(i > 0)
        def _():
            st_ref[...] = st_ref[...] + st

    @pl.when(i >= NRB)
    def _():
        mu, a, be = _bn_coeffs(st_ref, prm_ref, 4, 5)
        h = jnp.maximum((o2_ref[pl.ds((i - NRB) * RB, RB), :] - mu) * a + be,
                        0.0)
        b = bt_ref[0, 0, :]
        gid = lax.broadcasted_iota(jnp.int32, (G, RB), 0)
        oh = jnp.where(gid == b[None, :], 1.0, 0.0)
        ps = jnp.dot(oh, h, preferred_element_type=jnp.float32)
        cnt = jnp.broadcast_to(jnp.sum(oh, axis=1, keepdims=True), (G, H))

        @pl.when(i == NRB)
        def _():
            o_ref[...] = ps
            cnt_ref[...] = cnt

        @pl.when(i > NRB)
        def _():
            o_ref[...] = o_ref[...] + ps
            cnt_ref[...] = cnt_ref[...] + cnt

        @pl.when(i == 2 * NRB - 1)
        def _():
            o_ref[...] = o_ref[...] / jnp.maximum(cnt_ref[...], 1.0)


def _tc_tail(accpa, accpb, hpa, hpb, dcol, prm, bt):
    j = lambda i: jnp.where(i < NRB, i, i - NRB)
    return pl.pallas_call(
        _tail_body,
        grid=(2 * NRB,),
        in_specs=[pl.BlockSpec((RB, HH), lambda i: (j(i), 0)),
                  pl.BlockSpec((RB, HH), lambda i: (NRB + j(i), 0)),
                  pl.BlockSpec((RB, HH), lambda i: (j(i), 0)),
                  pl.BlockSpec((RB, HH), lambda i: (NRB + j(i), 0)),
                  pl.BlockSpec((RB, HH), lambda i: (j(i), 0)),
                  pl.BlockSpec((RB, HH), lambda i: (j(i), 0)),
                  pl.BlockSpec((RB, 1), lambda i: (j(i), 0)),
                  pl.BlockSpec((8, 128), lambda i: (0, 0)),
                  pl.BlockSpec((1, 1, RB), lambda i: (j(i), 0, 0))],
        out_specs=pl.BlockSpec((G, H), lambda i: (0, 0)),
        out_shape=jax.ShapeDtypeStruct((G, H), jnp.float32),
        scratch_shapes=[pltpu.VMEM((NP, H), jnp.float32),
                        pltpu.VMEM((8, H), jnp.float32),
                        pltpu.VMEM((G, H), jnp.float32)],
    )(accpa, accpa, accpb, accpb, hpa, hpb, dcol, prm, bt)


def kernel(x, ei, batch, W1, b1, g1, be1, W2, b2, g2, be2):
    src = ei[0].astype(jnp.int32)
    dst = ei[1].astype(jnp.int32)
    srcp = jnp.concatenate(
        [src, jnp.zeros((EP - E,), jnp.int32)]).reshape(NW, NBLK, BLK)
    dstp = jnp.concatenate(
        [dst, jnp.full((EP - E,), N, jnp.int32)]).reshape(NW, NBLK, BLK)
    x_pad = jnp.pad(x, ((0, NP - N), (0, 0)))
    bt = jnp.concatenate(
        [batch.astype(jnp.int32),
         jnp.full((NP - N,), G, jnp.int32)]).reshape(NRB, 1, RB)
    idr = jnp.arange(640, dtype=jnp.int32).reshape(5, 128)
    prm = jnp.pad(jnp.stack([b1, g1, be1, b2, g2, be2,
                             jnp.zeros_like(b1), jnp.zeros_like(b1)]),
                  ((0, 0), (0, 128 - H)))

    degp = _sc_deg(dst, idr)
    mm1 = _tc_mm(x_pad, W1)
    dinv = _tc_dinv(degp.reshape(2, 80, 128))
    dcol = dinv.reshape(NP)[:, None]

    h1pa, h1pb = _tc_scale(mm1, dcol)
    a1a, a1b = _sc_agg(h1pa, h1pb, srcp, dstp)
    h2pa, h2pb = _tc_mid(a1a, a1b, h1pa, h1pb, dcol, prm, W2, brow=0)
    a2a, a2b = _sc_agg(h2pa, h2pb, srcp, dstp)
    return _tc_tail(a2a, a2b, h2pa, h2pb, dcol, prm, bt)


# speedup vs baseline: 1.0407x; 1.0407x over previous
import functools

import jax
import jax.numpy as jnp
from jax import lax
from jax.experimental import pallas as pl
from jax.experimental.pallas import tpu as pltpu
from jax.experimental.pallas import tpu_sc as plsc

N = 10000
E = 320000
F = 128
H = 64
G = 16
EPS = 1e-5

NC, NS = 2, 16
NW = NC * NS
NP = 10240
EP = 327680
BLK = 128
NBLK = EP // (NW * BLK)
DPT = E // NW
RB = 2048
NRB = NP // RB

@functools.cache
def _sc_params():
    import dataclasses
    cp = pltpu.CompilerParams()
    if "needs_layout_passes" in pltpu.CompilerParams.__dataclass_fields__:
        cp = dataclasses.replace(cp, needs_layout_passes=False)
    if "use_tc_tiling_on_sc" in pltpu.CompilerParams.__dataclass_fields__:
        cp = dataclasses.replace(cp, use_tc_tiling_on_sc=False)
    return cp


@functools.cache
def _mesh():
    return plsc.VectorSubcoreMesh(
        core_axis_name="c", subcore_axis_name="s",
        num_cores=NC, num_subcores=NS)


def _sc_deg_body(dst_hbm, idr_hbm, out_hbm, idx_v, hist_v, idr_v, zv, acc_sh,
                 sem):
    c = lax.axis_index("c")
    s = lax.axis_index("s")
    wid = s * NC + c
    zero16 = jnp.zeros((16,), jnp.float32)

    @pl.loop(0, 640)
    def _zero_hist(i):
        hist_v[i, :] = zero16

    @pl.loop(0, 40)
    def _zero_zv(i):
        zv[i, :] = zero16

    pltpu.sync_copy(zv, acc_sh.at[pl.ds(s * 40, 40)])
    pltpu.sync_copy(dst_hbm.at[pl.ds(wid * DPT, DPT)], idx_v)
    pltpu.sync_copy(idr_hbm, idr_v)
    plsc.subcore_barrier()

    ones16 = jnp.ones((16,), jnp.float32)

    @pl.loop(0, DPT // 16)
    def _hist(i):
        nid = idx_v[pl.ds(i * 16, 16)]
        row = lax.shift_right_logical(nid, 4)
        col = lax.bitwise_and(nid, 15)
        plsc.addupdate_scatter(hist_v, [row, col], ones16)

    @pl.loop(0, 5)
    def _merge(j):
        pltpu.sync_copy(hist_v.at[pl.ds(j * 128, 128)],
                        acc_sh.at[idr_v.at[j]], add=True)

    plsc.subcore_barrier()
    pltpu.sync_copy(acc_sh.at[pl.ds(s * 40, 40)],
                    out_hbm.at[pl.ds(c * 640 + s * 40, 40)])


@jax.jit
def _sc_deg(dst, idr):
    return pl.kernel(
        _sc_deg_body,
        out_type=jax.ShapeDtypeStruct((2 * 640, 16), jnp.float32),
        mesh=_mesh(),
        compiler_params=_sc_params(),
        scratch_types=[
            pltpu.VMEM((DPT,), jnp.int32),
            pltpu.VMEM((640, 16), jnp.float32),
            pltpu.VMEM((5, 128), jnp.int32),
            pltpu.VMEM((40, 16), jnp.float32),
            pltpu.VMEM_SHARED((640, 16), jnp.float32),
            pltpu.SemaphoreType.DMA,
        ],
    )(dst, idr)


HH = H // 2


def _sc_agg_body(tabA, tabB, srcp_hbm, dstp_hbm, outA, outB,
                 sidx_v, didx_v, rows, semg, sems, acc_sh, tab_sh):
    c = lax.axis_index("c")
    s = lax.axis_index("s")
    wid = s * NC + c
    rpt = NP // NS
    NB = 4

    pltpu.sync_copy(srcp_hbm.at[wid], sidx_v)
    pltpu.sync_copy(dstp_hbm.at[wid], didx_v)

    for tab_hbm, out_hbm in ((tabA, outA), (tabB, outB)):
        pltpu.sync_copy(tab_hbm.at[pl.ds(s * rpt, rpt)],
                        tab_sh.at[pl.ds(s * rpt, rpt)])

        @pl.loop(0, BLK)
        def _zrow(i):
            @pl.loop(0, HH, step=16)
            def _zcol(k):
                rows[0, i, pl.ds(k, 16)] = jnp.zeros((16,), jnp.float32)

        @pl.loop(0, rpt, step=BLK)
        def _zacc(r):
            pltpu.sync_copy(rows.at[0], acc_sh.at[pl.ds(s * rpt + r, BLK)])

        plsc.subcore_barrier()

        def gather(k, b):
            pltpu.async_copy(tab_sh.at[sidx_v.at[k]], rows.at[b], semg[b])

        def wait_gather(k, b):
            pltpu.make_async_copy(tab_sh.at[sidx_v.at[k]], rows.at[b],
                                  semg[b]).wait()

        def scat(k, b):
            pltpu.async_copy(rows.at[b], acc_sh.at[didx_v.at[k]], sems[b],
                             add=True)

        def wait_scat(k, b):
            pltpu.make_async_copy(rows.at[b], acc_sh.at[didx_v.at[k]],
                                  sems[b]).wait()

        gather(0, 0)
        gather(1, 1)

        @pl.loop(0, NBLK, step=NB)
        def _edges(j):
            for b in range(NB):
                k = j + b
                gb = (b + 2) % NB

                @pl.when(k - 2 >= 0)
                def _():
                    wait_scat(k - 2, gb)

                @pl.when(k + 2 < NBLK)
                def _():
                    gather(k + 2, gb)

                wait_gather(k, b)
                scat(k, b)

        for k in (NBLK - 2, NBLK - 1):
            wait_scat(k, k % NB)

        plsc.subcore_barrier()
        pltpu.sync_copy(acc_sh.at[pl.ds(s * rpt, rpt)],
                        out_hbm.at[pl.ds(c * NP + s * rpt, rpt)])
        plsc.subcore_barrier()


@jax.jit
def _sc_agg(tabA, tabB, srcp, dstp):
    return pl.kernel(
        _sc_agg_body,
        out_type=(jax.ShapeDtypeStruct((2 * NP, HH), jnp.float32),
                  jax.ShapeDtypeStruct((2 * NP, HH), jnp.float32)),
        mesh=_mesh(),
        compiler_params=_sc_params(),
        scratch_types=[
            pltpu.VMEM((NBLK, BLK), jnp.int32),
            pltpu.VMEM((NBLK, BLK), jnp.int32),
            pltpu.VMEM((4, BLK, HH), jnp.float32),
            [pltpu.SemaphoreType.DMA] * 4,
            [pltpu.SemaphoreType.DMA] * 4,
            pltpu.VMEM_SHARED((NP, HH), jnp.float32),
            pltpu.VMEM_SHARED((NP, HH), jnp.float32),
        ],
    )(tabA, tabB, srcp, dstp)


def _mm_body(x_ref, w_ref, o_ref):
    o_ref[...] = jnp.dot(x_ref[...], w_ref[...],
                         preferred_element_type=jnp.float32)


def _tc_mm(x, w):
    m, k = x.shape
    _, n = w.shape
    return pl.pallas_call(
        _mm_body,
        grid=(m // RB,),
        in_specs=[pl.BlockSpec((RB, k), lambda i: (i, 0)),
                  pl.BlockSpec((k, n), lambda i: (0, 0))],
        out_specs=pl.BlockSpec((RB, n), lambda i: (i, 0)),
        out_shape=jax.ShapeDtypeStruct((m, n), jnp.float32),
    )(x, w)


def _dinv_body(dp_ref, o_ref):
    deg = dp_ref[0] + dp_ref[1] + 1.0
    r = lax.broadcasted_iota(jnp.int32, (80, 128), 0)
    cidx = lax.broadcasted_iota(jnp.int32, (80, 128), 1)
    nid = r * 128 + cidx
    o_ref[...] = jnp.where(nid < N, lax.rsqrt(deg), 0.0)


def _tc_dinv(degp):
    return pl.pallas_call(
        _dinv_body,
        out_shape=jax.ShapeDtypeStruct((80, 128), jnp.float32),
    )(degp)


def _scale_body(m_ref, d_ref, oa_ref, ob_ref):
    v = m_ref[...] * d_ref[...]
    oa_ref[...] = v[:, :HH]
    ob_ref[...] = v[:, HH:]


def _tc_scale(m, dcol):
    return pl.pallas_call(
        _scale_body,
        grid=(NRB,),
        in_specs=[pl.BlockSpec((RB, H), lambda i: (i, 0)),
                  pl.BlockSpec((RB, 1), lambda i: (i, 0))],
        out_specs=[pl.BlockSpec((RB, HH), lambda i: (i, 0)),
                   pl.BlockSpec((RB, HH), lambda i: (i, 0))],
        out_shape=[jax.ShapeDtypeStruct((NP, HH), jnp.float32),
                   jax.ShapeDtypeStruct((NP, HH), jnp.float32)],
    )(m, dcol)


def _accum_out(aa0, aa1, ab0, ab1, hpa, hpb, d, b, i):
    agg = jnp.concatenate([aa0 + aa1 + hpa, ab0 + ab1 + hpb], axis=1)
    v = agg * d + b
    rid = lax.broadcasted_iota(jnp.int32, (RB, 1), 0) + i * RB
    return jnp.where(rid < N, v, 0.0)


def _bn_coeffs(st_ref, prm_ref, grow, berow):
    mu = st_ref[0:1, :] * (1.0 / N)
    var = st_ref[1:2, :] * (1.0 / N) - mu * mu
    istd = lax.rsqrt(var + EPS)
    g = prm_ref[grow:grow + 1, :H]
    be = prm_ref[berow:berow + 1, :H]
    return mu, istd * g, be


def _mid_body(aa0_ref, aa1_ref, ab0_ref, ab1_ref, hpa_ref, hpb_ref, d_ref,
              prm_ref, w_ref, oa_ref, ob_ref, o1_ref, st_ref, *, brow):
    i = pl.program_id(0)

    @pl.when(i < NRB)
    def _():
        b = prm_ref[brow:brow + 1, :H]
        v = _accum_out(aa0_ref[...], aa1_ref[...], ab0_ref[...], ab1_ref[...],
                       hpa_ref[...], hpb_ref[...], d_ref[...], b, i)
        o1_ref[pl.ds(i * RB, RB), :] = v
        srow = jnp.sum(v, axis=0, keepdims=True)
        qrow = jnp.sum(v * v, axis=0, keepdims=True)
        st = jnp.concatenate(
            [srow, qrow, jnp.zeros((6, H), jnp.float32)], axis=0)

        @pl.when(i == 0)
        def _():
            st_ref[...] = st

        @pl.when(i > 0)
        def _():
            st_ref[...] = st_ref[...] + st

    @pl.when(i >= NRB)
    def _():
        mu, a, be = _bn_coeffs(st_ref, prm_ref, brow + 1, brow + 2)
        h = jnp.maximum((o1_ref[pl.ds((i - NRB) * RB, RB), :] - mu) * a + be,
                        0.0)
        v = jnp.dot(h, w_ref[...],
                    preferred_element_type=jnp.float32) * d_ref[...]
        oa_ref[...] = v[:, :HH]
        ob_ref[...] = v[:, HH:]


def _tc_mid(accpa, accpb, hpa, hpb, dcol, prm, w, brow):
    j = lambda i: jnp.where(i < NRB, i, i - NRB)
    jp = lambda i: jnp.where(i < NRB, i, 0)
    jo = lambda i: jnp.where(i < NRB, 0, i - NRB)
    return pl.pallas_call(
        functools.partial(_mid_body, brow=brow),
        grid=(2 * NRB,),
        in_specs=[pl.BlockSpec((RB, HH), lambda i: (jp(i), 0)),
                  pl.BlockSpec((RB, HH), lambda i: (NRB + jp(i), 0)),
                  pl.BlockSpec((RB, HH), lambda i: (jp(i), 0)),
                  pl.BlockSpec((RB, HH), lambda i: (NRB + jp(i), 0)),
                  pl.BlockSpec((RB, HH), lambda i: (jp(i), 0)),
                  pl.BlockSpec((RB, HH), lambda i: (jp(i), 0)),
                  pl.BlockSpec((RB, 1), lambda i: (j(i), 0)),
                  pl.BlockSpec((8, 128), lambda i: (0, 0)),
                  pl.BlockSpec((H, H), lambda i: (0, 0))],
        out_specs=[pl.BlockSpec((RB, HH), lambda i: (jo(i), 0)),
                   pl.BlockSpec((RB, HH), lambda i: (jo(i), 0))],
        out_shape=[jax.ShapeDtypeStruct((NP, HH), jnp.float32),
                   jax.ShapeDtypeStruct((NP, HH), jnp.float32)],
        scratch_shapes=[pltpu.VMEM((NP, H), jnp.float32),
                        pltpu.VMEM((8, H), jnp.float32)],
    )(accpa, accpa, accpb, accpb, hpa, hpb, dcol, prm, w)


def _tail_body(aa0_ref, aa1_ref, ab0_ref, ab1_ref, hpa_ref, hpb_ref, d_ref,
               prm_ref, bt_ref, o_ref, o2_ref, st_ref, cnt_ref):
    i = pl.program_id(0)

    @pl.when(i < NRB)
    def _():
        b = prm_ref[3:4, :H]
        v = _accum_out(aa0_ref[...], aa1_ref[...], ab0_ref[...], ab1_ref[...],
                       hpa_ref[...], hpb_ref[...], d_ref[...], b, i)
        o2_ref[pl.ds(i * RB, RB), :] = v
        srow = jnp.sum(v, axis=0, keepdims=True)
        qrow = jnp.sum(v * v, axis=0, keepdims=True)
        st = jnp.concatenate(
            [srow, qrow, jnp.zeros((6, H), jnp.float32)], axis=0)

        @pl.when(i == 0)
        def _():
            st_ref[...] = st

        @pl.when(i > 0)
        def _():
            st_ref[...] = st_ref[...] + st

    @pl.when(i >= NRB)
    def _():
        mu, a, be = _bn_coeffs(st_ref, prm_ref, 4, 5)
        h = jnp.maximum((o2_ref[pl.ds((i - NRB) * RB, RB), :] - mu) * a + be,
                        0.0)
        b = bt_ref[0, 0, :]
        gid = lax.broadcasted_iota(jnp.int32, (G, RB), 0)
        oh = jnp.where(gid == b[None, :], 1.0, 0.0)
        ps = jnp.dot(oh, h, preferred_element_type=jnp.float32)
        cnt = jnp.broadcast_to(jnp.sum(oh, axis=1, keepdims=True), (G, H))

        @pl.when(i == NRB)
        def _():
            o_ref[...] = ps
            cnt_ref[...] = cnt

        @pl.when(i > NRB)
        def _():
            o_ref[...] = o_ref[...] + ps
            cnt_ref[...] = cnt_ref[...] + cnt

        @pl.when(i == 2 * NRB - 1)
        def _():
            o_ref[...] = o_ref[...] / jnp.maximum(cnt_ref[...], 1.0)


def _tc_tail(accpa, accpb, hpa, hpb, dcol, prm, bt):
    j = lambda i: jnp.where(i < NRB, i, i - NRB)
    jp = lambda i: jnp.where(i < NRB, i, 0)
    return pl.pallas_call(
        _tail_body,
        grid=(2 * NRB,),
        in_specs=[pl.BlockSpec((RB, HH), lambda i: (jp(i), 0)),
                  pl.BlockSpec((RB, HH), lambda i: (NRB + jp(i), 0)),
                  pl.BlockSpec((RB, HH), lambda i: (jp(i), 0)),
                  pl.BlockSpec((RB, HH), lambda i: (NRB + jp(i), 0)),
                  pl.BlockSpec((RB, HH), lambda i: (jp(i), 0)),
                  pl.BlockSpec((RB, HH), lambda i: (jp(i), 0)),
                  pl.BlockSpec((RB, 1), lambda i: (jp(i), 0)),
                  pl.BlockSpec((8, 128), lambda i: (0, 0)),
                  pl.BlockSpec((1, 1, RB), lambda i: (j(i), 0, 0))],
        out_specs=pl.BlockSpec((G, H), lambda i: (0, 0)),
        out_shape=jax.ShapeDtypeStruct((G, H), jnp.float32),
        scratch_shapes=[pltpu.VMEM((NP, H), jnp.float32),
                        pltpu.VMEM((8, H), jnp.float32),
                        pltpu.VMEM((G, H), jnp.float32)],
    )(accpa, accpa, accpb, accpb, hpa, hpb, dcol, prm, bt)


def kernel(x, ei, batch, W1, b1, g1, be1, W2, b2, g2, be2):
    src = ei[0].astype(jnp.int32)
    dst = ei[1].astype(jnp.int32)
    srcp = jnp.concatenate(
        [src, jnp.zeros((EP - E,), jnp.int32)]).reshape(NW, NBLK, BLK)
    dstp = jnp.concatenate(
        [dst, jnp.full((EP - E,), N, jnp.int32)]).reshape(NW, NBLK, BLK)
    x_pad = jnp.pad(x, ((0, NP - N), (0, 0)))
    bt = jnp.concatenate(
        [batch.astype(jnp.int32),
         jnp.full((NP - N,), G, jnp.int32)]).reshape(NRB, 1, RB)
    idr = jnp.arange(640, dtype=jnp.int32).reshape(5, 128)
    prm = jnp.pad(jnp.stack([b1, g1, be1, b2, g2, be2,
                             jnp.zeros_like(b1), jnp.zeros_like(b1)]),
                  ((0, 0), (0, 128 - H)))

    degp = _sc_deg(dst, idr)
    mm1 = _tc_mm(x_pad, W1)
    dinv = _tc_dinv(degp.reshape(2, 80, 128))
    dcol = dinv.reshape(NP)[:, None]

    h1pa, h1pb = _tc_scale(mm1, dcol)
    a1a, a1b = _sc_agg(h1pa, h1pb, srcp, dstp)
    h2pa, h2pb = _tc_mid(a1a, a1b, h1pa, h1pb, dcol, prm, W2, brow=0)
    a2a, a2b = _sc_agg(h2pa, h2pb, srcp, dstp)
    return _tc_tail(a2a, a2b, h2pa, h2pb, dcol, prm, bt)


# speedup vs baseline: 1.0444x; 1.0036x over previous
import functools

import jax
import jax.numpy as jnp
from jax import lax
from jax.experimental import pallas as pl
from jax.experimental.pallas import tpu as pltpu
from jax.experimental.pallas import tpu_sc as plsc

N = 10000
E = 320000
F = 128
H = 64
G = 16
EPS = 1e-5

NC, NS = 2, 16
NW = NC * NS
NP = 10240
EP = 327680
BLK = 128
NBLK = EP // (NW * BLK)
DPT = E // NW
RB = 2048
NRB = NP // RB

@functools.cache
def _sc_params():
    import dataclasses
    cp = pltpu.CompilerParams()
    if "needs_layout_passes" in pltpu.CompilerParams.__dataclass_fields__:
        cp = dataclasses.replace(cp, needs_layout_passes=False)
    if "use_tc_tiling_on_sc" in pltpu.CompilerParams.__dataclass_fields__:
        cp = dataclasses.replace(cp, use_tc_tiling_on_sc=False)
    return cp


@functools.cache
def _mesh():
    return plsc.VectorSubcoreMesh(
        core_axis_name="c", subcore_axis_name="s",
        num_cores=NC, num_subcores=NS)


def _sc_deg_body(dst_hbm, idr_hbm, out_hbm, idx_v, hist_v, idr_v, zv, acc_sh,
                 sem):
    c = lax.axis_index("c")
    s = lax.axis_index("s")
    wid = s * NC + c
    zero16 = jnp.zeros((16,), jnp.float32)

    @pl.loop(0, 640)
    def _zero_hist(i):
        hist_v[i, :] = zero16

    @pl.loop(0, 40)
    def _zero_zv(i):
        zv[i, :] = zero16

    pltpu.sync_copy(zv, acc_sh.at[pl.ds(s * 40, 40)])
    pltpu.sync_copy(dst_hbm.at[pl.ds(wid * DPT, DPT)], idx_v)
    pltpu.sync_copy(idr_hbm, idr_v)
    plsc.subcore_barrier()

    ones16 = jnp.ones((16,), jnp.float32)

    @pl.loop(0, DPT // 16)
    def _hist(i):
        nid = idx_v[pl.ds(i * 16, 16)]
        row = lax.shift_right_logical(nid, 4)
        col = lax.bitwise_and(nid, 15)
        plsc.addupdate_scatter(hist_v, [row, col], ones16)

    @pl.loop(0, 5)
    def _merge(j):
        pltpu.sync_copy(hist_v.at[pl.ds(j * 128, 128)],
                        acc_sh.at[idr_v.at[j]], add=True)

    plsc.subcore_barrier()
    pltpu.sync_copy(acc_sh.at[pl.ds(s * 40, 40)],
                    out_hbm.at[pl.ds(c * 640 + s * 40, 40)])


@jax.jit
def _sc_deg(dst, idr):
    return pl.kernel(
        _sc_deg_body,
        out_type=jax.ShapeDtypeStruct((2 * 640, 16), jnp.float32),
        mesh=_mesh(),
        compiler_params=_sc_params(),
        scratch_types=[
            pltpu.VMEM((DPT,), jnp.int32),
            pltpu.VMEM((640, 16), jnp.float32),
            pltpu.VMEM((5, 128), jnp.int32),
            pltpu.VMEM((40, 16), jnp.float32),
            pltpu.VMEM_SHARED((640, 16), jnp.float32),
            pltpu.SemaphoreType.DMA,
        ],
    )(dst, idr)


HH = H // 2


def _sc_agg_body(tabA, tabB, srcp_hbm, dstp_hbm, outA, outB,
                 sidx_v, didx_v, rows, zb, semg, sems, acc_sh, tab_sh):
    c = lax.axis_index("c")
    s = lax.axis_index("s")
    wid = s * NC + c
    rpt = NP // NS
    NB = 4

    pltpu.sync_copy(srcp_hbm.at[wid], sidx_v)
    pltpu.sync_copy(dstp_hbm.at[wid], didx_v)

    @pl.loop(0, BLK)
    def _zrow(i):
        @pl.loop(0, HH, step=16)
        def _zcol(k):
            zb[i, pl.ds(k, 16)] = jnp.zeros((16,), jnp.float32)

    for tab_hbm, out_hbm in ((tabA, outA), (tabB, outB)):
        tab_sp = tab_sh
        pltpu.sync_copy(tab_hbm.at[pl.ds(s * rpt, rpt)],
                        tab_sp.at[pl.ds(s * rpt, rpt)])

        @pl.loop(0, rpt, step=BLK)
        def _zacc(r):
            pltpu.sync_copy(zb, acc_sh.at[pl.ds(s * rpt + r, BLK)])

        plsc.subcore_barrier()

        def gather(k, b, tab_sp=tab_sp):
            pltpu.async_copy(tab_sp.at[sidx_v.at[k]], rows.at[b], semg[b])

        def wait_gather(k, b, tab_sp=tab_sp):
            pltpu.make_async_copy(tab_sp.at[sidx_v.at[k]], rows.at[b],
                                  semg[b]).wait()

        def scat(k, b):
            pltpu.async_copy(rows.at[b], acc_sh.at[didx_v.at[k]], sems[b],
                             add=True)

        def wait_scat(k, b):
            pltpu.make_async_copy(rows.at[b], acc_sh.at[didx_v.at[k]],
                                  sems[b]).wait()

        gather(0, 0)
        gather(1, 1)

        @pl.loop(0, NBLK, step=NB)
        def _edges(j):
            for b in range(NB):
                k = j + b
                gb = (b + 2) % NB

                @pl.when(k - 2 >= 0)
                def _():
                    wait_scat(k - 2, gb)

                @pl.when(k + 2 < NBLK)
                def _():
                    gather(k + 2, gb)

                wait_gather(k, b)
                scat(k, b)

        for k in (NBLK - 2, NBLK - 1):
            wait_scat(k, k % NB)

        plsc.subcore_barrier()
        pltpu.sync_copy(acc_sh.at[pl.ds(s * rpt, rpt)],
                        out_hbm.at[pl.ds(c * NP + s * rpt, rpt)])
        plsc.subcore_barrier()


@jax.jit
def _sc_agg(tabA, tabB, srcp, dstp):
    return pl.kernel(
        _sc_agg_body,
        out_type=(jax.ShapeDtypeStruct((2 * NP, HH), jnp.float32),
                  jax.ShapeDtypeStruct((2 * NP, HH), jnp.float32)),
        mesh=_mesh(),
        compiler_params=_sc_params(),
        scratch_types=[
            pltpu.VMEM((NBLK, BLK), jnp.int32),
            pltpu.VMEM((NBLK, BLK), jnp.int32),
            pltpu.VMEM((4, BLK, HH), jnp.float32),
            pltpu.VMEM((BLK, HH), jnp.float32),
            [pltpu.SemaphoreType.DMA] * 4,
            [pltpu.SemaphoreType.DMA] * 4,
            pltpu.VMEM_SHARED((NP, HH), jnp.float32),
            pltpu.VMEM_SHARED((NP, HH), jnp.float32),
        ],
    )(tabA, tabB, srcp, dstp)


def _mm_body(x_ref, w_ref, o_ref):
    o_ref[...] = jnp.dot(x_ref[...], w_ref[...],
                         preferred_element_type=jnp.float32)


def _tc_mm(x, w):
    m, k = x.shape
    _, n = w.shape
    return pl.pallas_call(
        _mm_body,
        grid=(m // RB,),
        in_specs=[pl.BlockSpec((RB, k), lambda i: (i, 0)),
                  pl.BlockSpec((k, n), lambda i: (0, 0))],
        out_specs=pl.BlockSpec((RB, n), lambda i: (i, 0)),
        out_shape=jax.ShapeDtypeStruct((m, n), jnp.float32),
    )(x, w)


def _dinv_body(dp_ref, o_ref):
    deg = dp_ref[0] + dp_ref[1] + 1.0
    r = lax.broadcasted_iota(jnp.int32, (80, 128), 0)
    cidx = lax.broadcasted_iota(jnp.int32, (80, 128), 1)
    nid = r * 128 + cidx
    o_ref[...] = jnp.where(nid < N, lax.rsqrt(deg), 0.0)


def _tc_dinv(degp):
    return pl.pallas_call(
        _dinv_body,
        out_shape=jax.ShapeDtypeStruct((80, 128), jnp.float32),
    )(degp)


def _scale_body(m_ref, d_ref, oa_ref, ob_ref):
    v = m_ref[...] * d_ref[...]
    oa_ref[...] = v[:, :HH]
    ob_ref[...] = v[:, HH:]


def _tc_scale(m, dcol):
    return pl.pallas_call(
        _scale_body,
        grid=(NRB,),
        in_specs=[pl.BlockSpec((RB, H), lambda i: (i, 0)),
                  pl.BlockSpec((RB, 1), lambda i: (i, 0))],
        out_specs=[pl.BlockSpec((RB, HH), lambda i: (i, 0)),
                   pl.BlockSpec((RB, HH), lambda i: (i, 0))],
        out_shape=[jax.ShapeDtypeStruct((NP, HH), jnp.float32),
                   jax.ShapeDtypeStruct((NP, HH), jnp.float32)],
    )(m, dcol)


def _accum_out(aa0, aa1, ab0, ab1, hpa, hpb, d, b, i):
    agg = jnp.concatenate([aa0 + aa1 + hpa, ab0 + ab1 + hpb], axis=1)
    v = agg * d + b
    rid = lax.broadcasted_iota(jnp.int32, (RB, 1), 0) + i * RB
    return jnp.where(rid < N, v, 0.0)


def _bn_coeffs(st_ref, prm_ref, grow, berow):
    mu = st_ref[0:1, :] * (1.0 / N)
    var = st_ref[1:2, :] * (1.0 / N) - mu * mu
    istd = lax.rsqrt(var + EPS)
    g = prm_ref[grow:grow + 1, :H]
    be = prm_ref[berow:berow + 1, :H]
    return mu, istd * g, be


def _mid_body(aa0_ref, aa1_ref, ab0_ref, ab1_ref, hpa_ref, hpb_ref, d_ref,
              prm_ref, w_ref, oa_ref, ob_ref, o1_ref, st_ref, *, brow):
    i = pl.program_id(0)

    @pl.when(i < NRB)
    def _():
        b = prm_ref[brow:brow + 1, :H]
        v = _accum_out(aa0_ref[...], aa1_ref[...], ab0_ref[...], ab1_ref[...],
                       hpa_ref[...], hpb_ref[...], d_ref[...], b, i)
        o1_ref[pl.ds(i * RB, RB), :] = v
        srow = jnp.sum(v, axis=0, keepdims=True)
        qrow = jnp.sum(v * v, axis=0, keepdims=True)
        st = jnp.concatenate(
            [srow, qrow, jnp.zeros((6, H), jnp.float32)], axis=0)

        @pl.when(i == 0)
        def _():
            st_ref[...] = st

        @pl.when(i > 0)
        def _():
            st_ref[...] = st_ref[...] + st

    @pl.when(i >= NRB)
    def _():
        mu, a, be = _bn_coeffs(st_ref, prm_ref, brow + 1, brow + 2)
        h = jnp.maximum((o1_ref[pl.ds((i - NRB) * RB, RB), :] - mu) * a + be,
                        0.0)
        v = jnp.dot(h, w_ref[...],
                    preferred_element_type=jnp.float32) * d_ref[...]
        oa_ref[...] = v[:, :HH]
        ob_ref[...] = v[:, HH:]


def _tc_mid(accpa, accpb, hpa, hpb, dcol, prm, w, brow):
    j = lambda i: jnp.where(i < NRB, i, i - NRB)
    jp = lambda i: jnp.where(i < NRB, i, 0)
    jo = lambda i: jnp.where(i < NRB, 0, i - NRB)
    return pl.pallas_call(
        functools.partial(_mid_body, brow=brow),
        grid=(2 * NRB,),
        in_specs=[pl.BlockSpec((RB, HH), lambda i: (jp(i), 0)),
                  pl.BlockSpec((RB, HH), lambda i: (NRB + jp(i), 0)),
                  pl.BlockSpec((RB, HH), lambda i: (jp(i), 0)),
                  pl.BlockSpec((RB, HH), lambda i: (NRB + jp(i), 0)),
                  pl.BlockSpec((RB, HH), lambda i: (jp(i), 0)),
                  pl.BlockSpec((RB, HH), lambda i: (jp(i), 0)),
                  pl.BlockSpec((RB, 1), lambda i: (j(i), 0)),
                  pl.BlockSpec((8, 128), lambda i: (0, 0)),
                  pl.BlockSpec((H, H), lambda i: (0, 0))],
        out_specs=[pl.BlockSpec((RB, HH), lambda i: (jo(i), 0)),
                   pl.BlockSpec((RB, HH), lambda i: (jo(i), 0))],
        out_shape=[jax.ShapeDtypeStruct((NP, HH), jnp.float32),
                   jax.ShapeDtypeStruct((NP, HH), jnp.float32)],
        scratch_shapes=[pltpu.VMEM((NP, H), jnp.float32),
                        pltpu.VMEM((8, H), jnp.float32)],
    )(accpa, accpa, accpb, accpb, hpa, hpb, dcol, prm, w)


def _tail_body(aa0_ref, aa1_ref, ab0_ref, ab1_ref, hpa_ref, hpb_ref, d_ref,
               prm_ref, bt_ref, o_ref, o2_ref, st_ref, cnt_ref):
    i = pl.program_id(0)

    @pl.when(i < NRB)
    def _():
        b = prm_ref[3:4, :H]
        v = _accum_out(aa0_ref[...], aa1_ref[...], ab0_ref[...], ab1_ref[...],
                       hpa_ref[...], hpb_ref[...], d_ref[...], b, i)
        o2_ref[pl.ds(i * RB, RB), :] = v
        srow = jnp.sum(v, axis=0, keepdims=True)
        qrow = jnp.sum(v * v, axis=0, keepdims=True)
        st = jnp.concatenate(
            [srow, qrow, jnp.zeros((6, H), jnp.float32)], axis=0)

        @pl.when(i == 0)
        def _():
            st_ref[...] = st

        @pl.when(i > 0)
        def _():
            st_ref[...] = st_ref[...] + st

    @pl.when(i >= NRB)
    def _():
        mu, a, be = _bn_coeffs(st_ref, prm_ref, 4, 5)
        h = jnp.maximum((o2_ref[pl.ds((i - NRB) * RB, RB), :] - mu) * a + be,
                        0.0)
        b = bt_ref[0, 0, :]
        gid = lax.broadcasted_iota(jnp.int32, (G, RB), 0)
        oh = jnp.where(gid == b[None, :], 1.0, 0.0)
        ps = jnp.dot(oh, h, preferred_element_type=jnp.float32)
        cnt = jnp.broadcast_to(jnp.sum(oh, axis=1, keepdims=True), (G, H))

        @pl.when(i == NRB)
        def _():
            o_ref[...] = ps
            cnt_ref[...] = cnt

        @pl.when(i > NRB)
        def _():
            o_ref[...] = o_ref[...] + ps
            cnt_ref[...] = cnt_ref[...] + cnt

        @pl.when(i == 2 * NRB - 1)
        def _():
            o_ref[...] = o_ref[...] / jnp.maximum(cnt_ref[...], 1.0)


def _tc_tail(accpa, accpb, hpa, hpb, dcol, prm, bt):
    j = lambda i: jnp.where(i < NRB, i, i - NRB)
    jp = lambda i: jnp.where(i < NRB, i, 0)
    return pl.pallas_call(
        _tail_body,
        grid=(2 * NRB,),
        in_specs=[pl.BlockSpec((RB, HH), lambda i: (jp(i), 0)),
                  pl.BlockSpec((RB, HH), lambda i: (NRB + jp(i), 0)),
                  pl.BlockSpec((RB, HH), lambda i: (jp(i), 0)),
                  pl.BlockSpec((RB, HH), lambda i: (NRB + jp(i), 0)),
                  pl.BlockSpec((RB, HH), lambda i: (jp(i), 0)),
                  pl.BlockSpec((RB, HH), lambda i: (jp(i), 0)),
                  pl.BlockSpec((RB, 1), lambda i: (jp(i), 0)),
                  pl.BlockSpec((8, 128), lambda i: (0, 0)),
                  pl.BlockSpec((1, 1, RB), lambda i: (j(i), 0, 0))],
        out_specs=pl.BlockSpec((G, H), lambda i: (0, 0)),
        out_shape=jax.ShapeDtypeStruct((G, H), jnp.float32),
        scratch_shapes=[pltpu.VMEM((NP, H), jnp.float32),
                        pltpu.VMEM((8, H), jnp.float32),
                        pltpu.VMEM((G, H), jnp.float32)],
    )(accpa, accpa, accpb, accpb, hpa, hpb, dcol, prm, bt)


def kernel(x, ei, batch, W1, b1, g1, be1, W2, b2, g2, be2):
    src = ei[0].astype(jnp.int32)
    dst = ei[1].astype(jnp.int32)
    srcp = jnp.concatenate(
        [src, jnp.zeros((EP - E,), jnp.int32)]).reshape(NW, NBLK, BLK)
    dstp = jnp.concatenate(
        [dst, jnp.full((EP - E,), N, jnp.int32)]).reshape(NW, NBLK, BLK)
    x_pad = jnp.pad(x, ((0, NP - N), (0, 0)))
    bt = jnp.concatenate(
        [batch.astype(jnp.int32),
         jnp.full((NP - N,), G, jnp.int32)]).reshape(NRB, 1, RB)
    idr = jnp.arange(640, dtype=jnp.int32).reshape(5, 128)
    prm = jnp.pad(jnp.stack([b1, g1, be1, b2, g2, be2,
                             jnp.zeros_like(b1), jnp.zeros_like(b1)]),
                  ((0, 0), (0, 128 - H)))

    degp = _sc_deg(dst, idr)
    mm1 = _tc_mm(x_pad, W1)
    dinv = _tc_dinv(degp.reshape(2, 80, 128))
    dcol = dinv.reshape(NP)[:, None]

    h1pa, h1pb = _tc_scale(mm1, dcol)
    a1a, a1b = _sc_agg(h1pa, h1pb, srcp, dstp)
    h2pa, h2pb = _tc_mid(a1a, a1b, h1pa, h1pb, dcol, prm, W2, brow=0)
    a2a, a2b = _sc_agg(h2pa, h2pb, srcp, dstp)
    return _tc_tail(a2a, a2b, h2pa, h2pb, dcol, prm, bt)
